# baseline (speedup 1.0000x reference)
import dataclasses
import functools

import jax
import jax.numpy as jnp
from jax import lax
from jax.experimental import pallas as pl
from jax.experimental.pallas import tpu as pltpu
from jax.experimental.pallas import tpu_sc as plsc

_NUM_EXPERTS = 64
_TOP_K = 8
_LANES = 16
_NW = 32


def _compiler_params():
    cp = pltpu.CompilerParams()
    if "needs_layout_passes" in pltpu.CompilerParams.__dataclass_fields__:
        cp = dataclasses.replace(cp, needs_layout_passes=False)
    return cp


@functools.partial(jax.jit, static_argnames=("n",))
def _reorder(scores_flat, experts_flat, n):
    chunk = n // _NW
    n_vregs = chunk // _LANES
    rows = chunk // 128
    group = 8 if rows % 8 == 0 else 1
    mesh = plsc.VectorSubcoreMesh(core_axis_name="c", subcore_axis_name="s")
    cp = _compiler_params()

    @functools.partial(
        pl.kernel,
        out_type=[
            jax.ShapeDtypeStruct((n,), jnp.int32),
            jax.ShapeDtypeStruct((_NW, _NUM_EXPERTS), jnp.int32),
        ],
        mesh=mesh,
        compiler_params=cp,
        scratch_types=[
            pltpu.VMEM((chunk,), jnp.int32),
            pltpu.VMEM((chunk,), jnp.int32),
            pltpu.VMEM((chunk,), jnp.int32),
            pltpu.VMEM((chunk,), jnp.int32),
            pltpu.VMEM((chunk,), jnp.int32),
            pltpu.VMEM((chunk,), jnp.int32),
            pltpu.VMEM((_NUM_EXPERTS,), jnp.int32),
            pltpu.VMEM((_NUM_EXPERTS,), jnp.int32),
        ],
    )
    def run1(experts_hbm, pack_hbm, hist_hbm,
             e_chunk, sebuf, rsbuf, tailbuf, lposbuf, packbuf,
             counters, local_off):
        cid = lax.axis_index("c")
        sid = lax.axis_index("s")
        wid = cid * 16 + sid
        base = wid * chunk
        lane = lax.iota(jnp.int32, _LANES)

        pltpu.sync_copy(experts_hbm.at[pl.ds(base, chunk)], e_chunk)

        @plsc.parallel_loop(0, n_vregs, 1, unroll=8)
        def _(v):
            off = v * _LANES
            e = e_chunk[pl.ds(off, _LANES)]
            key = e * _LANES + lane
            skey, slane = plsc.sort_key_val(key, lane)
            se = skey >> 4
            sebuf[pl.ds(off, _LANES)] = se
            prev = plsc.load_gather(
                sebuf, [jnp.maximum(off + lane - 1, off)]
            )
            nxt = plsc.load_gather(
                sebuf, [jnp.minimum(off + lane + 1, off + _LANES - 1)]
            )
            head = (lane == 0) | (se != prev)
            tail = (lane == _LANES - 1) | (se != nxt)
            segstart = plsc.cummax(jnp.where(head, lane, 0))
            rank = lane - segstart
            rsbuf[pl.ds(off, _LANES)] = (rank << 4) | slane
            tailbuf[pl.ds(off, _LANES)] = jnp.where(tail, 1, 0)

        for j in range(_NUM_EXPERTS // _LANES):
            counters[pl.ds(j * _LANES, _LANES)] = jnp.zeros(
                (_LANES,), jnp.int32
            )

        @pl.loop(0, n_vregs)
        def _(v):
            off = v * _LANES
            cs = pl.ds(off, _LANES)
            se = sebuf[cs]
            rs = rsbuf[cs]
            cold = plsc.load_gather(counters, [se])
            lpos = cold + (rs >> 4)
            plsc.store_scatter(
                counters, [se], lpos + 1, mask=(tailbuf[cs] == 1)
            )
            lposbuf[cs] = lpos

        lcarry = jnp.int32(0)
        for c in range(_NUM_EXPERTS // _LANES):
            cnt = counters[pl.ds(c * _LANES, _LANES)]
            csum = plsc.cumsum(cnt)
            local_off[pl.ds(c * _LANES, _LANES)] = (csum - cnt) + lcarry
            lcarry = lcarry + jnp.sum(cnt)

        @plsc.parallel_loop(0, n_vregs, 1, unroll=8)
        def _(v):
            off = v * _LANES
            cs = pl.ds(off, _LANES)
            se = sebuf[cs]
            lpos = lposbuf[cs]
            orig = off + (rsbuf[cs] & 15)
            word = (se << 26) | (lpos << 13) | orig
            localpos = plsc.load_gather(local_off, [se]) + lpos
            plsc.store_scatter(packbuf, [localpos], word)

        pltpu.sync_copy(packbuf, pack_hbm.at[pl.ds(base, chunk)])
        pltpu.sync_copy(counters, hist_hbm.at[wid])

    @functools.partial(
        pl.kernel,
        out_type=[
            jax.ShapeDtypeStruct((n,), jnp.float32),
            jax.ShapeDtypeStruct((n,), jnp.int32),
            jax.ShapeDtypeStruct((_NUM_EXPERTS,), jnp.float32),
        ],
        mesh=mesh,
        compiler_params=cp,
        scratch_types=[
            pltpu.VMEM((chunk,), jnp.float32),
            pltpu.VMEM((chunk,), jnp.int32),
            pltpu.VMEM((chunk,), jnp.int32),
            pltpu.VMEM((chunk,), jnp.float32),
            pltpu.VMEM((chunk,), jnp.int32),
            pltpu.VMEM((_NW, _NUM_EXPERTS), jnp.int32),
            pltpu.VMEM((_NUM_EXPERTS,), jnp.int32),
            pltpu.VMEM((_NUM_EXPERTS,), jnp.float32),
            [pltpu.VMEM((128,), jnp.int32) for _ in range(group)],
            pltpu.SemaphoreType.DMA,
            pltpu.SemaphoreType.DMA,
        ],
    )
    def run2(scores_hbm, pack_hbm, hist_hbm,
             out_scores, out_tok, out_counts,
             s_chunk, packch, posbuf, scorebuf, tokbuf,
             allhist, starts, countsf, idxbufs, sem_a, sem_b):
        cid = lax.axis_index("c")
        sid = lax.axis_index("s")
        wid = cid * 16 + sid
        base = wid * chunk
        lane = lax.iota(jnp.int32, _LANES)

        pltpu.sync_copy(scores_hbm.at[pl.ds(base, chunk)], s_chunk)
        pltpu.sync_copy(pack_hbm.at[pl.ds(base, chunk)], packch)
        pltpu.sync_copy(hist_hbm, allhist)

        carry = jnp.int32(0)
        for c in range(_NUM_EXPERTS // _LANES):
            tot = jnp.zeros((_LANES,), jnp.int32)
            mine = jnp.zeros((_LANES,), jnp.int32)
            for w in range(_NW):
                h = allhist[w, pl.ds(c * _LANES, _LANES)]
                tot = tot + h
                mine = mine + jnp.where(
                    jnp.full((_LANES,), w, jnp.int32) < wid, h, 0
                )
            csum = plsc.cumsum(tot)
            starts[pl.ds(c * _LANES, _LANES)] = (csum - tot) + mine + carry
            countsf[pl.ds(c * _LANES, _LANES)] = tot.astype(jnp.float32)
            carry = carry + jnp.sum(tot)

        @pl.when(wid == 0)
        def _():
            pltpu.sync_copy(countsf, out_counts)

        @plsc.parallel_loop(0, n_vregs, 1, unroll=8)
        def _(v):
            cs = pl.ds(v * _LANES, _LANES)
            pk = packch[cs]
            se = lax.shift_right_logical(pk, 26)
            lpos = lax.shift_right_logical(pk, 13) & 0x1FFF
            orig = pk & 0x1FFF
            st = plsc.load_gather(starts, [se])
            posbuf[cs] = st + lpos
            scorebuf[cs] = plsc.load_gather(s_chunk, [orig])
            tokbuf[cs] = (base + orig) >> 3

        def fill(idxbuf, r):
            for j in range(128 // _LANES):
                idxbuf[pl.ds(j * _LANES, _LANES)] = posbuf[
                    pl.ds(r * 128 + j * _LANES, _LANES)
                ]

        def fire(idxbuf, r):
            c1 = pltpu.async_copy(
                scorebuf.at[pl.ds(r * 128, 128)],
                out_scores.at[idxbuf],
                sem_a,
            )
            c2 = pltpu.async_copy(
                tokbuf.at[pl.ds(r * 128, 128)],
                out_tok.at[idxbuf],
                sem_b,
            )
            return (c1, c2)

        @pl.loop(0, rows, step=group)
        def _(r0):
            copies = []
            for b in range(group):
                fill(idxbufs[b], r0 + b)
                copies.extend(fire(idxbufs[b], r0 + b))
            for cp_ in copies:
                cp_.wait()

    pack, hist = run1(experts_flat)
    out = run2(scores_flat, pack, hist)
    return (out[0], out[1], out[2])


def kernel(top_scores, selected_experts_indices):
    n = top_scores.shape[0] * top_scores.shape[1]
    scores_flat = top_scores.reshape(-1)
    experts_flat = selected_experts_indices.reshape(-1)
    return _reorder(scores_flat, experts_flat, n)


# speedup vs baseline: 13.3563x; 13.3563x over previous
import dataclasses
import functools

import jax
import jax.numpy as jnp
from jax import lax
from jax.experimental import pallas as pl
from jax.experimental.pallas import tpu as pltpu
from jax.experimental.pallas import tpu_sc as plsc

_NUM_EXPERTS = 64
_TOP_K = 8
_LANES = 16
_NW = 32


def _compiler_params():
    cp = pltpu.CompilerParams()
    if "needs_layout_passes" in pltpu.CompilerParams.__dataclass_fields__:
        cp = dataclasses.replace(cp, needs_layout_passes=False)
    return cp


@functools.partial(jax.jit, static_argnames=("n",))
def _reorder(scores_flat, experts_flat, n):
    chunk = n // _NW
    n_vregs = chunk // _LANES
    rows = chunk // 128
    group = 8 if rows % 8 == 0 else 1
    mesh = plsc.VectorSubcoreMesh(core_axis_name="c", subcore_axis_name="s")
    cp = _compiler_params()

    @functools.partial(
        pl.kernel,
        out_type=[
            jax.ShapeDtypeStruct((n,), jnp.int32),
            jax.ShapeDtypeStruct((_NW, _NUM_EXPERTS), jnp.int32),
        ],
        mesh=mesh,
        compiler_params=cp,
        scratch_types=[
            pltpu.VMEM((chunk,), jnp.int32),
            pltpu.VMEM((chunk,), jnp.int32),
            pltpu.VMEM((chunk,), jnp.int32),
            pltpu.VMEM((chunk,), jnp.int32),
            pltpu.VMEM((chunk,), jnp.int32),
            pltpu.VMEM((_NUM_EXPERTS,), jnp.int32),
        ],
    )
    def run1(experts_hbm, pack_hbm, hist_hbm,
             e_chunk, sebuf, rsbuf, tailbuf, packbuf, counters):
        cid = lax.axis_index("c")
        sid = lax.axis_index("s")
        wid = cid * 16 + sid
        base = wid * chunk
        lane = lax.iota(jnp.int32, _LANES)

        pltpu.sync_copy(experts_hbm.at[pl.ds(base, chunk)], e_chunk)

        @plsc.parallel_loop(0, n_vregs, 1, unroll=8)
        def _(v):
            off = v * _LANES
            e = e_chunk[pl.ds(off, _LANES)]
            key = e * _LANES + lane
            skey, slane = plsc.sort_key_val(key, lane)
            se = skey >> 4
            sebuf[pl.ds(off, _LANES)] = se
            prev = plsc.load_gather(
                sebuf, [jnp.maximum(off + lane - 1, off)]
            )
            nxt = plsc.load_gather(
                sebuf, [jnp.minimum(off + lane + 1, off + _LANES - 1)]
            )
            head = (lane == 0) | (se != prev)
            tail = (lane == _LANES - 1) | (se != nxt)
            segstart = plsc.cummax(jnp.where(head, lane, 0))
            rank = lane - segstart
            rsbuf[pl.ds(off, _LANES)] = (rank << 4) | slane
            tailbuf[pl.ds(off, _LANES)] = jnp.where(tail, 1, 0)

        for j in range(_NUM_EXPERTS // _LANES):
            counters[pl.ds(j * _LANES, _LANES)] = jnp.zeros(
                (_LANES,), jnp.int32
            )

        @pl.loop(0, n_vregs)
        def _(v):
            off = v * _LANES
            cs = pl.ds(off, _LANES)
            se = sebuf[cs]
            rs = rsbuf[cs]
            cold = plsc.load_gather(counters, [se])
            lpos = cold + (rs >> 4)
            plsc.store_scatter(
                counters, [se], lpos + 1, mask=(tailbuf[cs] == 1)
            )
            packbuf[cs] = (se << 18) | (lpos << 4) | (rs & 15)

        pltpu.sync_copy(packbuf, pack_hbm.at[pl.ds(base, chunk)])
        pltpu.sync_copy(counters, hist_hbm.at[wid])

    chunk2 = n // 16
    n_vregs2 = chunk2 // _LANES
    rows2 = chunk2 // 128
    group2 = 8 if rows2 % 8 == 0 else 1
    mesh2 = plsc.VectorSubcoreMesh(
        core_axis_name="c", subcore_axis_name="s",
        num_cores=1, num_subcores=16,
    )

    @functools.partial(
        pl.kernel,
        out_type=[
            jax.ShapeDtypeStruct((n,), jnp.float32),
            jax.ShapeDtypeStruct((n,), jnp.int32),
            jax.ShapeDtypeStruct((_NUM_EXPERTS,), jnp.float32),
        ],
        mesh=mesh2,
        compiler_params=cp,
        scratch_types=[
            pltpu.VMEM((chunk2,), jnp.float32),
            pltpu.VMEM((chunk2,), jnp.int32),
            pltpu.VMEM((chunk2,), jnp.int32),
            pltpu.VMEM((chunk2,), jnp.float32),
            pltpu.VMEM((chunk2,), jnp.int32),
            pltpu.VMEM((_NW, _NUM_EXPERTS), jnp.int32),
            pltpu.VMEM((_NUM_EXPERTS,), jnp.int32),
            pltpu.VMEM((_NUM_EXPERTS,), jnp.int32),
            pltpu.VMEM((_NUM_EXPERTS,), jnp.float32),
            [pltpu.VMEM((128,), jnp.int32) for _ in range(group2)],
            pltpu.VMEM_SHARED((n,), jnp.float32),
            pltpu.VMEM_SHARED((n,), jnp.int32),
            pltpu.SemaphoreType.DMA,
            pltpu.SemaphoreType.DMA,
        ],
    )
    def run2(scores_hbm, pack_hbm, hist_hbm,
             out_scores, out_tok, out_counts,
             s_chunk, packch, posbuf, scorebuf, tokbuf,
             allhist, starts_a, starts_b, countsf, idxbufs,
             shared_s, shared_t, sem_a, sem_b):
        wid = lax.axis_index("s")
        base = wid * chunk2

        pltpu.sync_copy(scores_hbm.at[pl.ds(base, chunk2)], s_chunk)
        pltpu.sync_copy(pack_hbm.at[pl.ds(base, chunk2)], packch)
        pltpu.sync_copy(hist_hbm, allhist)

        carry = jnp.int32(0)
        for c in range(_NUM_EXPERTS // _LANES):
            tot = jnp.zeros((_LANES,), jnp.int32)
            mine_a = jnp.zeros((_LANES,), jnp.int32)
            mine_b = jnp.zeros((_LANES,), jnp.int32)
            for w in range(_NW):
                h = allhist[w, pl.ds(c * _LANES, _LANES)]
                tot = tot + h
                wv = jnp.full((_LANES,), w, jnp.int32)
                mine_a = mine_a + jnp.where(wv < 2 * wid, h, 0)
                mine_b = mine_b + jnp.where(wv < 2 * wid + 1, h, 0)
            csum = plsc.cumsum(tot)
            excl = (csum - tot) + carry
            starts_a[pl.ds(c * _LANES, _LANES)] = excl + mine_a
            starts_b[pl.ds(c * _LANES, _LANES)] = excl + mine_b
            countsf[pl.ds(c * _LANES, _LANES)] = tot.astype(jnp.float32)
            carry = carry + jnp.sum(tot)

        @pl.when(wid == 0)
        def _():
            pltpu.sync_copy(countsf, out_counts)

        for h in range(2):
            starts_ref = starts_a if h == 0 else starts_b
            hoff = h * (chunk2 // 2)
            base1 = base + hoff

            @plsc.parallel_loop(0, n_vregs2 // 2, 1, unroll=8)
            def _(v):
                cs = pl.ds(hoff + v * _LANES, _LANES)
                pk = packch[cs]
                se = pk >> 18
                lpos = (pk >> 4) & 0x3FFF
                slane = pk & 15
                st = plsc.load_gather(starts_ref, [se])
                posbuf[cs] = st + lpos
                scorebuf[cs] = plsc.load_gather(
                    s_chunk, [hoff + v * _LANES + slane]
                )
                tokbuf[cs] = (base1 + v * _LANES + slane) >> 3

        def fill(idxbuf, r):
            for j in range(128 // _LANES):
                idxbuf[pl.ds(j * _LANES, _LANES)] = posbuf[
                    pl.ds(r * 128 + j * _LANES, _LANES)
                ]

        def fire(idxbuf, r):
            c1 = pltpu.async_copy(
                scorebuf.at[pl.ds(r * 128, 128)],
                shared_s.at[idxbuf],
                sem_a,
            )
            c2 = pltpu.async_copy(
                tokbuf.at[pl.ds(r * 128, 128)],
                shared_t.at[idxbuf],
                sem_b,
            )
            return (c1, c2)

        @pl.loop(0, rows2, step=group2)
        def _(r0):
            copies = []
            for b in range(group2):
                fill(idxbufs[b], r0 + b)
                copies.extend(fire(idxbufs[b], r0 + b))
            for cp_ in copies:
                cp_.wait()

        plsc.subcore_barrier()
        pltpu.sync_copy(
            shared_s.at[pl.ds(base, chunk2)],
            out_scores.at[pl.ds(base, chunk2)],
        )
        pltpu.sync_copy(
            shared_t.at[pl.ds(base, chunk2)],
            out_tok.at[pl.ds(base, chunk2)],
        )

    pack, hist = run1(experts_flat)
    out = run2(scores_flat, pack, hist)
    return (out[0], out[1], out[2])


def kernel(top_scores, selected_experts_indices):
    n = top_scores.shape[0] * top_scores.shape[1]
    scores_flat = top_scores.reshape(-1)
    experts_flat = selected_experts_indices.reshape(-1)
    return _reorder(scores_flat, experts_flat, n)
